# Initial kernel scaffold; baseline (speedup 1.0000x reference)
import jax
import jax.numpy as jnp
from jax.experimental import pallas as pl


def kernel(lrtlist_g, scores_g, feat_zyx, W, b):
    raise NotImplementedError("write your pallas kernel here")


# speedup vs baseline: 7.0793x; 7.0793x over previous
import functools

import jax
import jax.numpy as jnp
from jax.experimental import pallas as pl
from jax.experimental.pallas import tpu as pltpu

B, N, C = 2, 20, 32
XD = YD = ZD = 32
PD = XD + 2
SX, SY = PD * PD, PD
LIN_RAW = PD * PD * PD
LIN = 39424
LOUT = 36992
ROWS = LOUT // 128
EPS = 1e-6
ANCHOR = 2.0


def _loss_kernel(params_ref, bias_ref, wf_ref, featp_ref, coords_ref, out_ref):
    cx = coords_ref[0]
    cy = coords_ref[1]
    cz = coords_ref[2]
    vm = coords_ref[3]

    f32 = jnp.float32
    cls_pos_num = f32(0.0)
    cls_neg_num = f32(0.0)
    reg_num = f32(0.0)
    pos_cnt = f32(0.0)
    neg_cnt = f32(0.0)

    for b in range(B):
        fb = featp_ref[b]
        acc = jnp.zeros((8, LOUT), f32)
        for k in range(27):
            i, j, kz = k // 9, (k // 3) % 3, k % 3
            sk = i * SX + j * SY + kz
            wk = wf_ref[k]
            tk = jax.lax.dot_general(
                wk, fb, (((1,), (0,)), ((), ())),
                preferred_element_type=f32)
            acc = acc + tk[:, sk:sk + LOUT]
        acc3 = acc.reshape(8, ROWS, 128)

        pos = jnp.zeros((ROWS, 128), f32)
        near = jnp.zeros((ROWS, 128), f32)
        gts = [jnp.zeros((ROWS, 128), f32) for _ in range(6)]
        for n in range(N):
            tx = params_ref[b, n, 0]
            ty = params_ref[b, n, 1]
            tz = params_ref[b, n, 2]
            ihx = params_ref[b, n, 3]
            ihy = params_ref[b, n, 4]
            ihz = params_ref[b, n, 5]
            dlx = params_ref[b, n, 6]
            dly = params_ref[b, n, 7]
            dlz = params_ref[b, n, 8]
            val = params_ref[b, n, 9]
            dx = tx - cx
            dy = ty - cy
            dz = tz - cz
            od = jnp.maximum(jnp.maximum(jnp.abs(dx) * ihx, jnp.abs(dy) * ihy),
                             jnp.abs(dz) * ihz)
            cover = jnp.where(od < 0.5, val, 0.0) * vm
            nearm = jnp.where(od < 0.8, val, 0.0) * vm
            w = cover * (1.0 - pos)
            gts[0] = gts[0] + w * (dx * (1.0 / ANCHOR))
            gts[1] = gts[1] + w * (dy * (1.0 / ANCHOR))
            gts[2] = gts[2] + w * (dz * (1.0 / ANCHOR))
            gts[3] = gts[3] + w * dlx
            gts[4] = gts[4] + w * dly
            gts[5] = gts[5] + w * dlz
            pos = jnp.maximum(pos, cover)
            near = jnp.maximum(near, nearm)

        pobj = jax.nn.sigmoid(acc3[0] + bias_ref[0, 0])
        negv = (1.0 - near) * vm
        cls_pos_num += jnp.sum(-pos * jnp.log(pobj + EPS))
        cls_neg_num += jnp.sum(-negv * jnp.log(1.0 - pobj + EPS))
        pos_cnt += jnp.sum(pos)
        neg_cnt += jnp.sum(negv)
        for ch in range(6):
            d = (acc3[ch + 1] + bias_ref[0, ch + 1]) - gts[ch]
            a = jnp.abs(d)
            sm = jnp.where(a < 1.0 / 9.0, 4.5 * d * d, a - 0.5 / 9.0)
            reg_num += jnp.sum(sm * pos)

    out_ref[0, 0] = (cls_pos_num / (pos_cnt + EPS)
                     + cls_neg_num / (neg_cnt + EPS)
                     + reg_num / (pos_cnt + EPS))


@functools.partial(jax.jit, static_argnames=())
def kernel(lrtlist_g, scores_g, feat_zyx, W, b):
    feat = jnp.transpose(feat_zyx, (0, 1, 4, 3, 2))
    featp = jnp.pad(feat, ((0, 0), (0, 0), (1, 1), (1, 1), (1, 1)))
    featp = featp.reshape(B, C, LIN_RAW)
    featp = jnp.pad(featp, ((0, 0), (0, 0), (0, LIN - LIN_RAW)))

    wf = jnp.transpose(W, (2, 3, 4, 0, 1)).reshape(27, 7, C)
    wf = jnp.pad(wf, ((0, 0), (0, 1), (0, 0)))

    lens = lrtlist_g[..., :3]
    t = lrtlist_g[..., 3:].reshape(B, N, 4, 4)[..., :3, 3]
    ih = 1.0 / (lens * 0.5 + 1e-5)
    dl = jnp.maximum(jnp.log(lens / ANCHOR), -1000000.0)
    params = jnp.concatenate(
        [t, ih, dl, scores_g[..., None]], axis=-1)
    bias = jnp.pad(b, (0, 1)).reshape(1, 8)

    g = jnp.arange(LOUT, dtype=jnp.int32)
    gx = g // SX
    gy = (g % SX) // SY
    gz = g % SY
    vm = ((gy < XD) & (gz < XD)).astype(jnp.float32)
    coords = jnp.stack([gx.astype(jnp.float32), gy.astype(jnp.float32),
                        gz.astype(jnp.float32), vm]).reshape(4, ROWS, 128)

    out = pl.pallas_call(
        _loss_kernel,
        out_shape=jax.ShapeDtypeStruct((1, 1), jnp.float32),
        in_specs=[
            pl.BlockSpec(memory_space=pltpu.SMEM),
            pl.BlockSpec(memory_space=pltpu.SMEM),
            pl.BlockSpec(memory_space=pltpu.VMEM),
            pl.BlockSpec(memory_space=pltpu.VMEM),
            pl.BlockSpec(memory_space=pltpu.VMEM),
        ],
        out_specs=pl.BlockSpec(memory_space=pltpu.SMEM),
    )(params, bias, wf, featp, coords)
    return out.reshape(())


# speedup vs baseline: 10.1187x; 1.4293x over previous
import jax
import jax.numpy as jnp
from jax.experimental import pallas as pl
from jax.experimental.pallas import tpu as pltpu

B, N, C = 2, 20, 32
XD = 32
PD = XD + 2
SZ, SY = PD * PD, PD
LIN_RAW = PD * PD * PD
LIN = 39424
LOUT = 36992
ROWS = LOUT // 128
EPS = 1e-6
ANCHOR = 2.0


def _loss_kernel(params_ref, bias_ref, wf_ref, featp_ref, coords_ref, out_ref):
    cx = coords_ref[0]
    cy = coords_ref[1]
    cz = coords_ref[2]
    vm = coords_ref[3]

    f32 = jnp.float32
    cls_pos_num = f32(0.0)
    cls_neg_num = f32(0.0)
    reg_num = f32(0.0)
    pos_cnt = f32(0.0)
    neg_cnt = f32(0.0)

    for b in range(B):
        fb = featp_ref[b]
        acc = jnp.zeros((8, LOUT), f32)
        for k in range(27):
            i, j, kz = k // 9, (k // 3) % 3, k % 3
            sk = i * SZ + j * SY + kz
            wk = wf_ref[k]
            tk = jax.lax.dot_general(
                wk, fb, (((1,), (0,)), ((), ())),
                preferred_element_type=f32)
            acc = acc + tk[:, sk:sk + LOUT]
        acc3 = acc.reshape(8, ROWS, 128)

        pos = jnp.zeros((ROWS, 128), f32)
        near = jnp.zeros((ROWS, 128), f32)
        gts = [jnp.zeros((ROWS, 128), f32) for _ in range(6)]
        for n in range(N):
            tx = params_ref[b, n, 0]
            ty = params_ref[b, n, 1]
            tz = params_ref[b, n, 2]
            ihx = params_ref[b, n, 3]
            ihy = params_ref[b, n, 4]
            ihz = params_ref[b, n, 5]
            dlx = params_ref[b, n, 6]
            dly = params_ref[b, n, 7]
            dlz = params_ref[b, n, 8]
            val = params_ref[b, n, 9]
            dx = tx - cx
            dy = ty - cy
            dz = tz - cz
            od = jnp.maximum(jnp.maximum(jnp.abs(dx) * ihx, jnp.abs(dy) * ihy),
                             jnp.abs(dz) * ihz)
            cover = jnp.where(od < 0.5, val, 0.0)
            nearm = jnp.where(od < 0.8, val, 0.0)
            w = cover * (1.0 - pos)
            gts[0] = gts[0] + w * (dx * (1.0 / ANCHOR))
            gts[1] = gts[1] + w * (dy * (1.0 / ANCHOR))
            gts[2] = gts[2] + w * (dz * (1.0 / ANCHOR))
            gts[3] = gts[3] + w * dlx
            gts[4] = gts[4] + w * dly
            gts[5] = gts[5] + w * dlz
            pos = jnp.maximum(pos, cover)
            near = jnp.maximum(near, nearm)

        pobj = jax.nn.sigmoid(acc3[0] + bias_ref[0, 0])
        negv = (1.0 - near) * vm
        cls_pos_num += jnp.sum(-pos * jnp.log(pobj + EPS))
        cls_neg_num += jnp.sum(-negv * jnp.log(1.0 - pobj + EPS))
        pos_cnt += jnp.sum(pos)
        neg_cnt += jnp.sum(negv)
        for ch in range(6):
            d = (acc3[ch + 1] + bias_ref[0, ch + 1]) - gts[ch]
            a = jnp.abs(d)
            sm = jnp.where(a < 1.0 / 9.0, 4.5 * d * d, a - 0.5 / 9.0)
            reg_num += jnp.sum(sm * pos)

    out_ref[0, 0] = (cls_pos_num / (pos_cnt + EPS)
                     + cls_neg_num / (neg_cnt + EPS)
                     + reg_num / (pos_cnt + EPS))


@jax.jit
def kernel(lrtlist_g, scores_g, feat_zyx, W, b):
    featp = jnp.pad(feat_zyx, ((0, 0), (0, 0), (1, 1), (1, 1), (1, 1)))
    featp = featp.reshape(B, C, LIN_RAW)
    featp = jnp.pad(featp, ((0, 0), (0, 0), (0, LIN - LIN_RAW)))
    featp = featp.astype(jnp.bfloat16)

    wf = jnp.transpose(W, (4, 3, 2, 0, 1)).reshape(27, 7, C)
    wf = jnp.pad(wf, ((0, 0), (0, 1), (0, 0))).astype(jnp.bfloat16)

    lens = lrtlist_g[..., :3]
    t = lrtlist_g[..., 3:].reshape(B, N, 4, 4)[..., :3, 3]
    ih = 1.0 / (lens * 0.5 + 1e-5)
    dl = jnp.maximum(jnp.log(lens / ANCHOR), -1000000.0)
    params = jnp.concatenate(
        [t, ih, dl, scores_g[..., None]], axis=-1)
    bias = jnp.pad(b, (0, 1)).reshape(1, 8)

    g = jnp.arange(LOUT, dtype=jnp.int32)
    gz = g // SZ
    gy = (g % SZ) // SY
    gx = g % SY
    inb = (gy < XD) & (gx < XD)
    vm = inb.astype(jnp.float32)
    big = jnp.float32(1e9)
    coords = jnp.stack([
        jnp.where(inb, gx.astype(jnp.float32), big),
        jnp.where(inb, gy.astype(jnp.float32), big),
        jnp.where(inb, gz.astype(jnp.float32), big),
        vm]).reshape(4, ROWS, 128)

    out = pl.pallas_call(
        _loss_kernel,
        out_shape=jax.ShapeDtypeStruct((1, 1), jnp.float32),
        in_specs=[
            pl.BlockSpec(memory_space=pltpu.SMEM),
            pl.BlockSpec(memory_space=pltpu.SMEM),
            pl.BlockSpec(memory_space=pltpu.VMEM),
            pl.BlockSpec(memory_space=pltpu.VMEM),
            pl.BlockSpec(memory_space=pltpu.VMEM),
        ],
        out_specs=pl.BlockSpec(memory_space=pltpu.SMEM),
    )(params, bias, wf, featp, coords)
    return out.reshape(())
